# baseline (speedup 1.0000x reference)
import functools

import jax
import jax.numpy as jnp
from jax import lax
from jax.experimental import pallas as pl
from jax.experimental.pallas import tpu as pltpu
from jax.experimental.pallas import tpu_sc as plsc

NUM_EMBEDDINGS = 1000000
EMBEDDING_DIM = 32

_INFO = plsc.get_sparse_core_info()
_NC, _NS = _INFO.num_cores, _INFO.num_subcores
_NW = _NC * _NS

_ROWS = 16384
_T = 26
_RPW = _ROWS // _NW
_NBUF = 2


def _make_kernel():
  mesh = plsc.VectorSubcoreMesh(core_axis_name="c", subcore_axis_name="s")

  @functools.partial(
      pl.kernel,
      out_type=jax.ShapeDtypeStruct((_ROWS, _T, EMBEDDING_DIM), jnp.float32),
      mesh=mesh,
      scratch_types=(
          [pltpu.VMEM((_RPW, _T), jnp.int32),
           pltpu.VMEM((_T, _RPW), jnp.int32)]
          + [pltpu.VMEM((_RPW, EMBEDDING_DIM), jnp.float32)] * _NBUF
          + [pltpu.SemaphoreType.DMA] * (2 * _NBUF)
      ),
      compiler_params=pltpu.CompilerParams(use_tc_tiling_on_sc=False, needs_layout_passes=False),
  )
  def emb_kernel(idx_hbm, table_hbm, out_hbm, idx_v, idx_t, *scratch):
    rows = scratch[:_NBUF]
    gsem = scratch[_NBUF:2 * _NBUF]
    osem = scratch[2 * _NBUF:]
    wid = lax.axis_index("s") * _NC + lax.axis_index("c")
    rbase = wid * _RPW
    pltpu.sync_copy(idx_hbm.at[pl.ds(rbase, _RPW)], idx_v)

    @pl.loop(0, _T)
    def _tr(t):
      tvec = jnp.full((16,), t, jnp.int32)
      for k in range(_RPW // 16):
        rvec = lax.iota(jnp.int32, 16) + k * 16
        idx_t[t, pl.ds(k * 16, 16)] = plsc.load_gather(idx_v, [rvec, tvec])

    def start_gather(t, b):
      pltpu.async_copy(table_hbm.at[idx_t.at[t]], rows[b], gsem[b])

    def wait_gather(b):
      pltpu.make_async_copy(table_hbm.at[idx_t.at[0]], rows[b], gsem[b]).wait()

    for b in range(_NBUF):
      start_gather(b, b)

    @pl.loop(0, _T)
    def _pos(t):
      b0 = lax.rem(t, _NBUF)
      for b in range(_NBUF):

        @pl.when(b0 == b)
        def _():
          wait_gather(b)
          pltpu.async_copy(
              rows[b], out_hbm.at[pl.ds(rbase, _RPW), t], osem[b]
          )
          pltpu.make_async_copy(
              rows[b], out_hbm.at[pl.ds(rbase, _RPW), 0], osem[b]
          ).wait()

          @pl.when(t < _T - _NBUF)
          def _():
            start_gather(t + _NBUF, b)

  return emb_kernel


_EMB = _make_kernel()


@jax.jit
def kernel(token_ids, E):
  return _EMB(token_ids, E)
